# Initial kernel scaffold; baseline (speedup 1.0000x reference)
import functools

import jax
import jax.numpy as jnp
from jax import lax
from jax.experimental import pallas as pl
from jax.experimental.pallas import tpu as pltpu
from jax.experimental.pallas import tpu_sc as plsc

N = 10000
D = 128
E2 = 640000
NC = 2
NS = 16
NW = NC * NS
M = E2 // NW
CW = 125
NCHUNK = M // CW
RPT = N // NS

_mesh = plsc.VectorSubcoreMesh(core_axis_name="c", subcore_axis_name="s")
_f32 = jnp.float32


def _zero_rows(buf, nrows):
    def row(r, _):
        for j in range(D // 16):
            buf[r, pl.ds(j * 16, 16)] = jnp.zeros((16,), _f32)
        return ()
    lax.fori_loop(0, nrows, row, ())


@functools.partial(
    pl.kernel,
    out_type=(
        jax.ShapeDtypeStruct((N,), _f32),
        jax.ShapeDtypeStruct((N, D), _f32),
    ),
    mesh=_mesh,
    scratch_types=[
        pltpu.VMEM((NCHUNK, CW), jnp.int32),
        pltpu.VMEM((128,), _f32),
        pltpu.VMEM((640,), _f32),
        pltpu.VMEM((400,), _f32),
        pltpu.VMEM((400,), _f32),
        pltpu.VMEM((400, D), _f32),
        pltpu.VMEM((400, D), _f32),
        pltpu.VMEM_SHARED((10240,), _f32),
    ],
)
def _prep(dst_hbm, emb_hbm, dinv_hbm, g0_hbm,
          idxb, ones, zb, histb, dinvb, embb, g0b, hist):
    c = lax.axis_index("c")
    s = lax.axis_index("s")
    w = c * NS + s

    for i in range(40):
        zb[pl.ds(i * 16, 16)] = jnp.zeros((16,), _f32)
    for i in range(8):
        ones[pl.ds(i * 16, 16)] = jnp.full((16,), 1.0, _f32)
    pltpu.sync_copy(zb, hist.at[pl.ds(s * 640, 640)])
    plsc.subcore_barrier()

    for t in range(2):
        pltpu.sync_copy(dst_hbm.at[s * 2 + t], idxb)

        def hbody(j, _):
            pltpu.sync_copy(ones.at[pl.ds(0, CW)], hist.at[idxb.at[j]],
                            add=True)
            return ()
        lax.fori_loop(0, NCHUNK, hbody, ())
    plsc.subcore_barrier()

    @pl.when(w < 25)
    def _():
        base = w * 400
        pltpu.sync_copy(hist.at[pl.ds(base, 400)], histb)
        pltpu.sync_copy(emb_hbm.at[pl.ds(base, 400)], embb)
        for i in range(25):
            h = histb[pl.ds(i * 16, 16)]
            x = jnp.maximum(h, 1.0)
            bits = plsc.bitcast(x, jnp.int32)
            y = plsc.bitcast(jnp.int32(0x5F3759DF) - (bits >> 1), _f32)
            for _ in range(3):
                y = y * (1.5 - 0.5 * x * y * y)
            dinvb[pl.ds(i * 16, 16)] = jnp.where(h > 0.5, y, 0.0)

        def row(r, _):
            d = dinvb[r]
            for j in range(D // 16):
                g0b[r, pl.ds(j * 16, 16)] = d * embb[r, pl.ds(j * 16, 16)]
            return ()
        lax.fori_loop(0, 400, row, ())
        pltpu.sync_copy(dinvb, dinv_hbm.at[pl.ds(base, 400)])
        pltpu.sync_copy(g0b, g0_hbm.at[pl.ds(base, 400)])


@functools.partial(
    pl.kernel,
    out_type=(
        jax.ShapeDtypeStruct((N, D), _f32),
        jax.ShapeDtypeStruct((N, D), _f32),
    ),
    mesh=_mesh,
    scratch_types=[
        pltpu.VMEM((NCHUNK, CW), jnp.int32),
        pltpu.VMEM((NCHUNK, CW), jnp.int32),
        pltpu.VMEM((CW, D), _f32),
        pltpu.VMEM_SHARED((N, D), _f32),
    ],
)
def _prop(g_hbm, dst_hbm, src_hbm, p0_hbm, p1_hbm, dstb, srcb, buf, acc):
    c = lax.axis_index("c")
    s = lax.axis_index("s")
    w = c * NS + s

    _zero_rows(buf, CW)
    for k in range(RPT // CW):
        pltpu.sync_copy(buf, acc.at[pl.ds(s * RPT + k * CW, CW)])
    plsc.subcore_barrier()

    pltpu.sync_copy(dst_hbm.at[w], dstb)
    pltpu.sync_copy(src_hbm.at[w], srcb)

    def body(j, _):
        pltpu.sync_copy(g_hbm.at[srcb.at[j]], buf)
        pltpu.sync_copy(buf, acc.at[dstb.at[j]], add=True)
        return ()
    lax.fori_loop(0, NCHUNK, body, ())
    plsc.subcore_barrier()

    @pl.when(c == 0)
    def _():
        for k in range(RPT // CW):
            r = s * RPT + k * CW
            pltpu.sync_copy(acc.at[pl.ds(r, CW)], p0_hbm.at[pl.ds(r, CW)])

    @pl.when(c == 1)
    def _():
        for k in range(RPT // CW):
            r = s * RPT + k * CW
            pltpu.sync_copy(acc.at[pl.ds(r, CW)], p1_hbm.at[pl.ds(r, CW)])


@functools.partial(
    pl.kernel,
    out_type=jax.ShapeDtypeStruct((N, D), _f32),
    mesh=_mesh,
    scratch_types=[
        pltpu.VMEM((200, D), _f32),
        pltpu.VMEM((200, D), _f32),
        pltpu.VMEM((200,), _f32),
    ],
)
def _scale(p0_hbm, p1_hbm, dinv_hbm, g_hbm, b0, b1, db):
    c = lax.axis_index("c")
    s = lax.axis_index("s")
    w = c * NS + s

    def chunk(k):
        base = k * 200
        pltpu.sync_copy(p0_hbm.at[pl.ds(base, 200)], b0)
        pltpu.sync_copy(p1_hbm.at[pl.ds(base, 200)], b1)
        pltpu.sync_copy(dinv_hbm.at[pl.ds(base, 200)], db)

        def row(r, _):
            d = db[r]
            d2 = d * d
            for j in range(D // 16):
                sl = pl.ds(j * 16, 16)
                b0[r, sl] = d2 * (b0[r, sl] + b1[r, sl])
            return ()
        lax.fori_loop(0, 200, row, ())
        pltpu.sync_copy(b0, g_hbm.at[pl.ds(base, 200)])

    chunk(w)

    @pl.when(w < 18)
    def _():
        chunk(w + 32)


def _heads_body(u, p01, p11, p02, p12, p03, p13, dinv,
                cw1, cb1, cw2, cb2, aw1, ab1, aw2, ab2, sw1, sb1, sw2, sb2,
                churn, cat, sku, uf):
    psum = (p01[...] + p11[...] + p02[...] + p12[...] + p03[...] + p13[...])
    x = (u[...] + dinv[...] * psum) * 0.25
    uf[...] = x

    def head(w1, b1, w2, b2):
        h = jnp.maximum(
            jnp.dot(x, w1[...], preferred_element_type=jnp.float32) + b1[...],
            0.0)
        return jax.nn.sigmoid(
            jnp.dot(h, w2[...], preferred_element_type=jnp.float32) + b2[...])

    churn[...] = head(cw1, cb1, cw2, cb2)
    cat[...] = head(aw1, ab1, aw2, ab2)
    sku[...] = head(sw1, sb1, sw2, sb2)


_BU = 1000


def _row_spec(cols):
    return pl.BlockSpec((_BU, cols), lambda i: (i, 0))


def _full_spec(r, cols):
    return pl.BlockSpec((r, cols), lambda i: (0, 0))


def _heads(u, p01, p11, p02, p12, p03, p13, dinv2,
           cw1, cb1, cw2, cb2, aw1, ab1, aw2, ab2, sw1, sb1, sw2, sb2):
    nu = u.shape[0]
    return pl.pallas_call(
        _heads_body,
        grid=(nu // _BU,),
        in_specs=[
            _row_spec(D),
            _row_spec(D), _row_spec(D), _row_spec(D),
            _row_spec(D), _row_spec(D), _row_spec(D),
            _row_spec(1),
            _full_spec(D, 128), _full_spec(1, 128),
            _full_spec(128, 1), _full_spec(1, 1),
            _full_spec(D, 128), _full_spec(1, 128),
            _full_spec(128, 100), _full_spec(1, 100),
            _full_spec(D, 128), _full_spec(1, 128),
            _full_spec(128, 1000), _full_spec(1, 1000),
        ],
        out_specs=[
            _row_spec(1), _row_spec(100), _row_spec(1000), _row_spec(D),
        ],
        out_shape=[
            jax.ShapeDtypeStruct((nu, 1), _f32),
            jax.ShapeDtypeStruct((nu, 100), _f32),
            jax.ShapeDtypeStruct((nu, 1000), _f32),
            jax.ShapeDtypeStruct((nu, D), _f32),
        ],
    )(u, p01, p11, p02, p12, p03, p13, dinv2,
      cw1, cb1, cw2, cb2, aw1, ab1, aw2, ab2, sw1, sb1, sw2, sb2)


def kernel(user_emb_w, item_emb_w, churn_w1, churn_b1, churn_w2, churn_b2,
           cat_w1, cat_b1, cat_w2, cat_b2, sku_w1, sku_b1, sku_w2, sku_b2,
           edge_index):
    ei = edge_index.astype(jnp.int32)
    dst = jnp.concatenate([ei[0], ei[1]]).reshape(NW, NCHUNK, CW)
    src = jnp.concatenate([ei[1], ei[0]]).reshape(NW, NCHUNK, CW)
    emb = jnp.concatenate([user_emb_w, item_emb_w], axis=0)

    dinv, g0 = _prep(dst, emb)
    p01, p11 = _prop(g0, dst, src)
    g1 = _scale(p01, p11, dinv)
    p02, p12 = _prop(g1, dst, src)
    g2 = _scale(p02, p12, dinv)
    p03, p13 = _prop(g2, dst, src)

    nu = user_emb_w.shape[0]
    dinv2 = dinv[:nu].reshape(nu, 1)
    churn, cat, sku, uf = _heads(
        user_emb_w, p01, p11, p02, p12, p03, p13, dinv2,
        churn_w1, churn_b1.reshape(1, 128), churn_w2, churn_b2.reshape(1, 1),
        cat_w1, cat_b1.reshape(1, 128), cat_w2, cat_b2.reshape(1, 100),
        sku_w1, sku_b1.reshape(1, 128), sku_w2, sku_b2.reshape(1, 1000))
    return churn, cat, sku, uf


# speedup vs baseline: 21.4579x; 21.4579x over previous
import functools

import jax
import jax.numpy as jnp
from jax import lax
from jax.experimental import pallas as pl
from jax.experimental.pallas import tpu as pltpu
from jax.experimental.pallas import tpu_sc as plsc

N = 10000
D = 128
E2 = 640000
NC = 2
NS = 16
NW = NC * NS
M = E2 // NW
CW = 125
NCHUNK = M // CW
IG = 8
NPAD = 10240
RPT = NPAD // NS
RC = 80

_mesh = plsc.VectorSubcoreMesh(core_axis_name="c", subcore_axis_name="s")
_f32 = jnp.float32


def _zero_rows(buf, nrows):
    def row(r, _):
        for j in range(D // 16):
            buf[r, pl.ds(j * 16, 16)] = jnp.zeros((16,), _f32)
        return ()
    lax.fori_loop(0, nrows, row, ())


def _rsqrt16(h):
    x = jnp.maximum(h, 1.0)
    bits = lax.bitcast_convert_type(x, jnp.int32)
    y = lax.bitcast_convert_type(jnp.int32(0x5F3759DF) - (bits >> 1), _f32)
    for _ in range(3):
        y = y * (1.5 - 0.5 * x * y * y)
    return jnp.where(h > 0.5, y, 0.0)


@functools.partial(
    pl.kernel,
    out_type=(
        jax.ShapeDtypeStruct((N,), _f32),
        jax.ShapeDtypeStruct((N, D), _f32),
    ),
    mesh=_mesh,
    scratch_types=[
        pltpu.VMEM((IG, CW), jnp.int32),
        pltpu.VMEM((128,), _f32),
        pltpu.VMEM((640,), _f32),
        pltpu.VMEM((RC,), _f32),
        pltpu.VMEM((RC,), _f32),
        pltpu.VMEM((RC, D), _f32),
        pltpu.VMEM((RC, D), _f32),
        pltpu.VMEM_SHARED((10240,), _f32),
    ],
)
def _prep(dst_hbm, emb_hbm, dinv_hbm, g0_hbm,
          idxb, ones, zb, histb, dinvb, embb, g0b, hist):
    c = lax.axis_index("c")
    s = lax.axis_index("s")
    w = c * NS + s

    for i in range(40):
        zb[pl.ds(i * 16, 16)] = jnp.zeros((16,), _f32)
    for i in range(8):
        ones[pl.ds(i * 16, 16)] = jnp.full((16,), 1.0, _f32)
    pltpu.sync_copy(zb, hist.at[pl.ds(s * 640, 640)])
    plsc.subcore_barrier()

    base_row = s * 2 * NCHUNK

    def hbody(j8, _):
        pltpu.sync_copy(dst_hbm.at[pl.ds(base_row + j8 * IG, IG)], idxb)
        for jj in range(IG):
            pltpu.sync_copy(ones.at[pl.ds(0, CW)], hist.at[idxb.at[jj]],
                            add=True)
        return ()
    lax.fori_loop(0, 2 * NCHUNK // IG, hbody, ())
    plsc.subcore_barrier()

    for t in range(4):
        kk = w + 32 * t

        @pl.when(kk < 125)
        def _():
            base = kk * RC
            pltpu.sync_copy(hist.at[pl.ds(base, RC)], histb)
            pltpu.sync_copy(emb_hbm.at[pl.ds(base, RC)], embb)
            for i in range(RC // 16):
                h = histb[pl.ds(i * 16, 16)]
                dinvb[pl.ds(i * 16, 16)] = _rsqrt16(h)

            def row16(r16, _):
                dv = dinvb[pl.ds(r16 * 16, 16)]
                for i in range(16):
                    d = dv[i]
                    r = r16 * 16 + i
                    for j in range(D // 16):
                        sl = pl.ds(j * 16, 16)
                        g0b[r, sl] = d * embb[r, sl]
                return ()
            lax.fori_loop(0, RC // 16, row16, ())
            pltpu.sync_copy(dinvb, dinv_hbm.at[pl.ds(base, RC)])
            pltpu.sync_copy(g0b, g0_hbm.at[pl.ds(base, RC)])


@functools.partial(
    pl.kernel,
    out_type=(
        jax.ShapeDtypeStruct((NPAD, D), _f32),
        jax.ShapeDtypeStruct((NPAD, D), _f32),
    ),
    mesh=_mesh,
    scratch_types=[
        pltpu.VMEM((IG, CW), jnp.int32),
        pltpu.VMEM((IG, CW), jnp.int32),
        pltpu.VMEM((CW, D), _f32),
        pltpu.VMEM((RC, D), _f32),
        pltpu.VMEM_SHARED((NPAD, D), _f32),
    ],
)
def _prop(g_hbm, dst_hbm, src_hbm, p0_hbm, p1_hbm, dstb, srcb, buf, zbuf, acc):
    c = lax.axis_index("c")
    s = lax.axis_index("s")
    w = c * NS + s

    _zero_rows(zbuf, RC)
    for k in range(RPT // RC):
        pltpu.sync_copy(zbuf, acc.at[pl.ds(s * RPT + k * RC, RC)])
    plsc.subcore_barrier()

    base_row = w * NCHUNK

    def body(j8, _):
        pltpu.sync_copy(dst_hbm.at[pl.ds(base_row + j8 * IG, IG)], dstb)
        pltpu.sync_copy(src_hbm.at[pl.ds(base_row + j8 * IG, IG)], srcb)
        for jj in range(IG):
            pltpu.sync_copy(g_hbm.at[srcb.at[jj]], buf)
            pltpu.sync_copy(buf, acc.at[dstb.at[jj]], add=True)
        return ()
    lax.fori_loop(0, NCHUNK // IG, body, ())
    plsc.subcore_barrier()

    @pl.when(c == 0)
    def _():
        for k in range(RPT // RC):
            r = s * RPT + k * RC
            pltpu.sync_copy(acc.at[pl.ds(r, RC)], p0_hbm.at[pl.ds(r, RC)])

    @pl.when(c == 1)
    def _():
        for k in range(RPT // RC):
            r = s * RPT + k * RC
            pltpu.sync_copy(acc.at[pl.ds(r, RC)], p1_hbm.at[pl.ds(r, RC)])


@functools.partial(
    pl.kernel,
    out_type=jax.ShapeDtypeStruct((N, D), _f32),
    mesh=_mesh,
    scratch_types=[
        pltpu.VMEM((RC, D), _f32),
        pltpu.VMEM((RC, D), _f32),
        pltpu.VMEM((RC,), _f32),
    ],
)
def _scale(p0_hbm, p1_hbm, dinv_hbm, g_hbm, b0, b1, db):
    c = lax.axis_index("c")
    s = lax.axis_index("s")
    w = c * NS + s

    for t in range(4):
        kk = w + 32 * t

        @pl.when(kk < 125)
        def _():
            base = kk * RC
            pltpu.sync_copy(p0_hbm.at[pl.ds(base, RC)], b0)
            pltpu.sync_copy(p1_hbm.at[pl.ds(base, RC)], b1)
            pltpu.sync_copy(dinv_hbm.at[pl.ds(base, RC)], db)

            def row16(r16, _):
                dv = db[pl.ds(r16 * 16, 16)]
                for i in range(16):
                    d = dv[i]
                    d2 = d * d
                    r = r16 * 16 + i
                    for j in range(D // 16):
                        sl = pl.ds(j * 16, 16)
                        b0[r, sl] = d2 * (b0[r, sl] + b1[r, sl])
                return ()
            lax.fori_loop(0, RC // 16, row16, ())
            pltpu.sync_copy(b0, g_hbm.at[pl.ds(base, RC)])


def _heads_body(u, p01, p11, p02, p12, p03, p13, dinv,
                cw1, cb1, cw2, cb2, aw1, ab1, aw2, ab2, sw1, sb1, sw2, sb2,
                churn, cat, sku, uf):
    psum = (p01[...] + p11[...] + p02[...] + p12[...] + p03[...] + p13[...])
    x = (u[...] + dinv[...] * psum) * 0.25
    uf[...] = x

    def head(w1, b1, w2, b2):
        h = jnp.maximum(
            jnp.dot(x, w1[...], preferred_element_type=jnp.float32) + b1[...],
            0.0)
        return jax.nn.sigmoid(
            jnp.dot(h, w2[...], preferred_element_type=jnp.float32) + b2[...])

    churn[...] = head(cw1, cb1, cw2, cb2)
    cat[...] = head(aw1, ab1, aw2, ab2)
    sku[...] = head(sw1, sb1, sw2, sb2)


_BU = 1000


def _row_spec(cols):
    return pl.BlockSpec((_BU, cols), lambda i: (i, 0))


def _full_spec(r, cols):
    return pl.BlockSpec((r, cols), lambda i: (0, 0))


def _heads(u, p01, p11, p02, p12, p03, p13, dinv2,
           cw1, cb1, cw2, cb2, aw1, ab1, aw2, ab2, sw1, sb1, sw2, sb2):
    nu = u.shape[0]
    return pl.pallas_call(
        _heads_body,
        grid=(nu // _BU,),
        in_specs=[
            _row_spec(D),
            _row_spec(D), _row_spec(D), _row_spec(D),
            _row_spec(D), _row_spec(D), _row_spec(D),
            _row_spec(1),
            _full_spec(D, 128), _full_spec(1, 128),
            _full_spec(128, 1), _full_spec(1, 1),
            _full_spec(D, 128), _full_spec(1, 128),
            _full_spec(128, 100), _full_spec(1, 100),
            _full_spec(D, 128), _full_spec(1, 128),
            _full_spec(128, 1000), _full_spec(1, 1000),
        ],
        out_specs=[
            _row_spec(1), _row_spec(100), _row_spec(1000), _row_spec(D),
        ],
        out_shape=[
            jax.ShapeDtypeStruct((nu, 1), _f32),
            jax.ShapeDtypeStruct((nu, 100), _f32),
            jax.ShapeDtypeStruct((nu, 1000), _f32),
            jax.ShapeDtypeStruct((nu, D), _f32),
        ],
    )(u, p01, p11, p02, p12, p03, p13, dinv2,
      cw1, cb1, cw2, cb2, aw1, ab1, aw2, ab2, sw1, sb1, sw2, sb2)


def kernel(user_emb_w, item_emb_w, churn_w1, churn_b1, churn_w2, churn_b2,
           cat_w1, cat_b1, cat_w2, cat_b2, sku_w1, sku_b1, sku_w2, sku_b2,
           edge_index):
    ei = edge_index.astype(jnp.int32)
    dst = jnp.concatenate([ei[0], ei[1]]).reshape(NW * NCHUNK, CW)
    src = jnp.concatenate([ei[1], ei[0]]).reshape(NW * NCHUNK, CW)
    emb = jnp.concatenate([user_emb_w, item_emb_w], axis=0)

    dinv, g0 = _prep(dst, emb)
    p01, p11 = _prop(g0, dst, src)
    g1 = _scale(p01, p11, dinv)
    p02, p12 = _prop(g1, dst, src)
    g2 = _scale(p02, p12, dinv)
    p03, p13 = _prop(g2, dst, src)

    nu = user_emb_w.shape[0]
    dinv2 = dinv[:nu].reshape(nu, 1)
    churn, cat, sku, uf = _heads(
        user_emb_w, p01, p11, p02, p12, p03, p13, dinv2,
        churn_w1, churn_b1.reshape(1, 128), churn_w2, churn_b2.reshape(1, 1),
        cat_w1, cat_b1.reshape(1, 128), cat_w2, cat_b2.reshape(1, 100),
        sku_w1, sku_b1.reshape(1, 128), sku_w2, sku_b2.reshape(1, 1000))
    return churn, cat, sku, uf


# speedup vs baseline: 32.5349x; 1.5162x over previous
import functools

import jax
import jax.numpy as jnp
from jax import lax
from jax.experimental import pallas as pl
from jax.experimental.pallas import tpu as pltpu
from jax.experimental.pallas import tpu_sc as plsc

N = 10000
D = 128
E2 = 640000
NC = 2
NS = 16
NW = NC * NS
M = E2 // NW
CW = 125
NCHUNK = M // CW
IG = 8
PG = 32
NPAD = 10240
RPT = NPAD // NS
RC = 80

_mesh = plsc.VectorSubcoreMesh(core_axis_name="c", subcore_axis_name="s")
_f32 = jnp.float32


def _zero_rows(buf, nrows):
    def row(r, _):
        for j in range(D // 16):
            buf[r, pl.ds(j * 16, 16)] = jnp.zeros((16,), _f32)
        return ()
    lax.fori_loop(0, nrows, row, ())


def _rsqrt16(h):
    x = jnp.maximum(h, 1.0)
    bits = lax.bitcast_convert_type(x, jnp.int32)
    y = lax.bitcast_convert_type(jnp.int32(0x5F3759DF) - (bits >> 1), _f32)
    for _ in range(3):
        y = y * (1.5 - 0.5 * x * y * y)
    return jnp.where(h > 0.5, y, 0.0)


@functools.partial(
    pl.kernel,
    out_type=(
        jax.ShapeDtypeStruct((N,), _f32),
        jax.ShapeDtypeStruct((N, D), _f32),
    ),
    mesh=_mesh,
    scratch_types=[
        pltpu.VMEM((IG, CW), jnp.int32),
        pltpu.VMEM((128,), _f32),
        pltpu.VMEM((640,), _f32),
        pltpu.VMEM((RC,), _f32),
        pltpu.VMEM((RC,), _f32),
        pltpu.VMEM((RC, D), _f32),
        pltpu.VMEM((RC, D), _f32),
        pltpu.VMEM_SHARED((10240,), _f32),
    ],
)
def _prep(dst_hbm, emb_hbm, dinv_hbm, g0_hbm,
          idxb, ones, zb, histb, dinvb, embb, g0b, hist):
    c = lax.axis_index("c")
    s = lax.axis_index("s")
    w = c * NS + s

    for i in range(40):
        zb[pl.ds(i * 16, 16)] = jnp.zeros((16,), _f32)
    for i in range(8):
        ones[pl.ds(i * 16, 16)] = jnp.full((16,), 1.0, _f32)
    pltpu.sync_copy(zb, hist.at[pl.ds(s * 640, 640)])
    plsc.subcore_barrier()

    base_row = s * 2 * NCHUNK

    def hbody(j8, _):
        pltpu.sync_copy(dst_hbm.at[pl.ds(base_row + j8 * IG, IG)], idxb)
        for jj in range(IG):
            pltpu.sync_copy(ones.at[pl.ds(0, CW)], hist.at[idxb.at[jj]],
                            add=True)
        return ()
    lax.fori_loop(0, 2 * NCHUNK // IG, hbody, ())
    plsc.subcore_barrier()

    for t in range(4):
        kk = w + 32 * t

        @pl.when(kk < 125)
        def _():
            base = kk * RC
            pltpu.sync_copy(hist.at[pl.ds(base, RC)], histb)
            pltpu.sync_copy(emb_hbm.at[pl.ds(base, RC)], embb)
            for i in range(RC // 16):
                h = histb[pl.ds(i * 16, 16)]
                dinvb[pl.ds(i * 16, 16)] = _rsqrt16(h)

            def row16(r16, _):
                dv = dinvb[pl.ds(r16 * 16, 16)]
                for i in range(16):
                    d = dv[i]
                    r = r16 * 16 + i
                    for j in range(D // 16):
                        sl = pl.ds(j * 16, 16)
                        g0b[r, sl] = d * embb[r, sl]
                return ()
            lax.fori_loop(0, RC // 16, row16, ())
            pltpu.sync_copy(dinvb, dinv_hbm.at[pl.ds(base, RC)])
            pltpu.sync_copy(g0b, g0_hbm.at[pl.ds(base, RC)])


@functools.partial(
    pl.kernel,
    out_type=(
        jax.ShapeDtypeStruct((NPAD, D), _f32),
        jax.ShapeDtypeStruct((NPAD, D), _f32),
    ),
    mesh=_mesh,
    scratch_types=[
        pltpu.VMEM((PG, CW), jnp.int32),
        pltpu.VMEM((PG, CW), jnp.int32),
        pltpu.VMEM((CW, D), _f32),
        pltpu.VMEM((CW, D), _f32),
        pltpu.SemaphoreType.DMA,
        pltpu.SemaphoreType.DMA,
        pltpu.SemaphoreType.DMA,
        pltpu.SemaphoreType.DMA,
        pltpu.VMEM_SHARED((NPAD, D), _f32),
    ],
)
def _prop(g_hbm, dst_hbm, src_hbm, p0_hbm, p1_hbm,
          dstb, srcb, b0, b1, g0s, g1s, s0s, s1s, acc):
    c = lax.axis_index("c")
    s = lax.axis_index("s")
    w = c * NS + s

    _zero_rows(b0, CW)
    for k in range(RPT // RC):
        pltpu.sync_copy(b0.at[pl.ds(0, RC)],
                        acc.at[pl.ds(s * RPT + k * RC, RC)])
    plsc.subcore_barrier()

    base_row = w * NCHUNK
    bufs = (b0, b1)
    gsems = (g0s, g1s)
    ssems = (s0s, s1s)
    pend_g = [None, None]
    pend_s = [None, None]

    for g in range(NCHUNK // PG):
        pltpu.sync_copy(dst_hbm.at[pl.ds(base_row + g * PG, PG)], dstb)
        pltpu.sync_copy(src_hbm.at[pl.ds(base_row + g * PG, PG)], srcb)
        pend_g[0] = pltpu.async_copy(g_hbm.at[srcb.at[0]], b0, g0s)
        for jj in range(PG):
            p = jj % 2
            if jj + 1 < PG:
                p1 = (jj + 1) % 2
                if pend_s[p1] is not None:
                    pend_s[p1].wait()
                pend_g[p1] = pltpu.async_copy(
                    g_hbm.at[srcb.at[jj + 1]], bufs[p1], gsems[p1])
            pend_g[p].wait()
            pend_g[p] = None
            pend_s[p] = pltpu.async_copy(
                bufs[p], acc.at[dstb.at[jj]], ssems[p], add=True)
        for p in range(2):
            if pend_s[p] is not None:
                pend_s[p].wait()
                pend_s[p] = None
    plsc.subcore_barrier()

    @pl.when(c == 0)
    def _():
        for k in range(RPT // RC):
            r = s * RPT + k * RC
            pltpu.sync_copy(acc.at[pl.ds(r, RC)], p0_hbm.at[pl.ds(r, RC)])

    @pl.when(c == 1)
    def _():
        for k in range(RPT // RC):
            r = s * RPT + k * RC
            pltpu.sync_copy(acc.at[pl.ds(r, RC)], p1_hbm.at[pl.ds(r, RC)])


@functools.partial(
    pl.kernel,
    out_type=jax.ShapeDtypeStruct((N, D), _f32),
    mesh=_mesh,
    scratch_types=[
        pltpu.VMEM((RC, D), _f32),
        pltpu.VMEM((RC, D), _f32),
        pltpu.VMEM((RC,), _f32),
    ],
)
def _scale(p0_hbm, p1_hbm, dinv_hbm, g_hbm, b0, b1, db):
    c = lax.axis_index("c")
    s = lax.axis_index("s")
    w = c * NS + s

    for t in range(4):
        kk = w + 32 * t

        @pl.when(kk < 125)
        def _():
            base = kk * RC
            pltpu.sync_copy(p0_hbm.at[pl.ds(base, RC)], b0)
            pltpu.sync_copy(p1_hbm.at[pl.ds(base, RC)], b1)
            pltpu.sync_copy(dinv_hbm.at[pl.ds(base, RC)], db)

            def row16(r16, _):
                dv = db[pl.ds(r16 * 16, 16)]
                for i in range(16):
                    d = dv[i]
                    d2 = d * d
                    r = r16 * 16 + i
                    for j in range(D // 16):
                        sl = pl.ds(j * 16, 16)
                        b0[r, sl] = d2 * (b0[r, sl] + b1[r, sl])
                return ()
            lax.fori_loop(0, RC // 16, row16, ())
            pltpu.sync_copy(b0, g_hbm.at[pl.ds(base, RC)])


def _heads_body(u, p01, p11, p02, p12, p03, p13, dinv,
                cw1, cb1, cw2, cb2, aw1, ab1, aw2, ab2, sw1, sb1, sw2, sb2,
                churn, cat, sku, uf):
    psum = (p01[...] + p11[...] + p02[...] + p12[...] + p03[...] + p13[...])
    x = (u[...] + dinv[...] * psum) * 0.25
    uf[...] = x

    def head(w1, b1, w2, b2):
        h = jnp.maximum(
            jnp.dot(x, w1[...], preferred_element_type=jnp.float32) + b1[...],
            0.0)
        return jax.nn.sigmoid(
            jnp.dot(h, w2[...], preferred_element_type=jnp.float32) + b2[...])

    churn[...] = head(cw1, cb1, cw2, cb2)
    cat[...] = head(aw1, ab1, aw2, ab2)
    sku[...] = head(sw1, sb1, sw2, sb2)


_BU = 1000


def _row_spec(cols):
    return pl.BlockSpec((_BU, cols), lambda i: (i, 0))


def _full_spec(r, cols):
    return pl.BlockSpec((r, cols), lambda i: (0, 0))


def _heads(u, p01, p11, p02, p12, p03, p13, dinv2,
           cw1, cb1, cw2, cb2, aw1, ab1, aw2, ab2, sw1, sb1, sw2, sb2):
    nu = u.shape[0]
    return pl.pallas_call(
        _heads_body,
        grid=(nu // _BU,),
        in_specs=[
            _row_spec(D),
            _row_spec(D), _row_spec(D), _row_spec(D),
            _row_spec(D), _row_spec(D), _row_spec(D),
            _row_spec(1),
            _full_spec(D, 128), _full_spec(1, 128),
            _full_spec(128, 1), _full_spec(1, 1),
            _full_spec(D, 128), _full_spec(1, 128),
            _full_spec(128, 100), _full_spec(1, 100),
            _full_spec(D, 128), _full_spec(1, 128),
            _full_spec(128, 1000), _full_spec(1, 1000),
        ],
        out_specs=[
            _row_spec(1), _row_spec(100), _row_spec(1000), _row_spec(D),
        ],
        out_shape=[
            jax.ShapeDtypeStruct((nu, 1), _f32),
            jax.ShapeDtypeStruct((nu, 100), _f32),
            jax.ShapeDtypeStruct((nu, 1000), _f32),
            jax.ShapeDtypeStruct((nu, D), _f32),
        ],
    )(u, p01, p11, p02, p12, p03, p13, dinv2,
      cw1, cb1, cw2, cb2, aw1, ab1, aw2, ab2, sw1, sb1, sw2, sb2)


def kernel(user_emb_w, item_emb_w, churn_w1, churn_b1, churn_w2, churn_b2,
           cat_w1, cat_b1, cat_w2, cat_b2, sku_w1, sku_b1, sku_w2, sku_b2,
           edge_index):
    ei = edge_index.astype(jnp.int32)
    dst = jnp.concatenate([ei[0], ei[1]]).reshape(NW * NCHUNK, CW)
    src = jnp.concatenate([ei[1], ei[0]]).reshape(NW * NCHUNK, CW)
    emb = jnp.concatenate([user_emb_w, item_emb_w], axis=0)

    dinv, g0 = _prep(dst, emb)
    p01, p11 = _prop(g0, dst, src)
    g1 = _scale(p01, p11, dinv)
    p02, p12 = _prop(g1, dst, src)
    g2 = _scale(p02, p12, dinv)
    p03, p13 = _prop(g2, dst, src)

    nu = user_emb_w.shape[0]
    dinv2 = dinv[:nu].reshape(nu, 1)
    churn, cat, sku, uf = _heads(
        user_emb_w, p01, p11, p02, p12, p03, p13, dinv2,
        churn_w1, churn_b1.reshape(1, 128), churn_w2, churn_b2.reshape(1, 1),
        cat_w1, cat_b1.reshape(1, 128), cat_w2, cat_b2.reshape(1, 100),
        sku_w1, sku_b1.reshape(1, 128), sku_w2, sku_b2.reshape(1, 1000))
    return churn, cat, sku, uf


# speedup vs baseline: 33.6724x; 1.0350x over previous
import functools

import jax
import jax.numpy as jnp
from jax import lax
from jax.experimental import pallas as pl
from jax.experimental.pallas import tpu as pltpu
from jax.experimental.pallas import tpu_sc as plsc

N = 10000
D = 128
E2 = 640000
NC = 2
NS = 16
NW = NC * NS
M = E2 // NW
CW = 125
NCHUNK = M // CW
IG = 8
PG = 16
NPAD = 10240
RPT = NPAD // NS
RC = 80

_mesh = plsc.VectorSubcoreMesh(core_axis_name="c", subcore_axis_name="s")
_f32 = jnp.float32


def _zero_rows(buf, nrows):
    def row(r, _):
        for j in range(D // 16):
            buf[r, pl.ds(j * 16, 16)] = jnp.zeros((16,), _f32)
        return ()
    lax.fori_loop(0, nrows, row, ())


def _rsqrt16(h):
    x = jnp.maximum(h, 1.0)
    bits = lax.bitcast_convert_type(x, jnp.int32)
    y = lax.bitcast_convert_type(jnp.int32(0x5F3759DF) - (bits >> 1), _f32)
    for _ in range(3):
        y = y * (1.5 - 0.5 * x * y * y)
    return jnp.where(h > 0.5, y, 0.0)


@functools.partial(
    pl.kernel,
    out_type=(
        jax.ShapeDtypeStruct((N,), _f32),
        jax.ShapeDtypeStruct((N, D), _f32),
    ),
    mesh=_mesh,
    scratch_types=[
        pltpu.VMEM((IG, CW), jnp.int32),
        pltpu.VMEM((128,), _f32),
        pltpu.VMEM((640,), _f32),
        pltpu.VMEM((RC,), _f32),
        pltpu.VMEM((RC,), _f32),
        pltpu.VMEM((RC, D), _f32),
        pltpu.VMEM((RC, D), _f32),
        pltpu.VMEM_SHARED((10240,), _f32),
    ],
)
def _prep(dst_hbm, emb_hbm, dinv_hbm, g0_hbm,
          idxb, ones, zb, histb, dinvb, embb, g0b, hist):
    c = lax.axis_index("c")
    s = lax.axis_index("s")
    w = c * NS + s

    for i in range(40):
        zb[pl.ds(i * 16, 16)] = jnp.zeros((16,), _f32)
    for i in range(8):
        ones[pl.ds(i * 16, 16)] = jnp.full((16,), 1.0, _f32)
    pltpu.sync_copy(zb, hist.at[pl.ds(s * 640, 640)])
    plsc.subcore_barrier()

    base_row = s * 2 * NCHUNK

    def hbody(j8, _):
        pltpu.sync_copy(dst_hbm.at[pl.ds(base_row + j8 * IG, IG)], idxb)
        for jj in range(IG):
            pltpu.sync_copy(ones.at[pl.ds(0, CW)], hist.at[idxb.at[jj]],
                            add=True)
        return ()
    lax.fori_loop(0, 2 * NCHUNK // IG, hbody, ())
    plsc.subcore_barrier()

    for t in range(4):
        kk = w + 32 * t

        @pl.when(kk < 125)
        def _():
            base = kk * RC
            pltpu.sync_copy(hist.at[pl.ds(base, RC)], histb)
            pltpu.sync_copy(emb_hbm.at[pl.ds(base, RC)], embb)
            for i in range(RC // 16):
                h = histb[pl.ds(i * 16, 16)]
                dinvb[pl.ds(i * 16, 16)] = _rsqrt16(h)

            def row16(r16, _):
                dv = dinvb[pl.ds(r16 * 16, 16)]
                for i in range(16):
                    d = dv[i]
                    r = r16 * 16 + i
                    for j in range(D // 16):
                        sl = pl.ds(j * 16, 16)
                        g0b[r, sl] = d * embb[r, sl]
                return ()
            lax.fori_loop(0, RC // 16, row16, ())
            pltpu.sync_copy(dinvb, dinv_hbm.at[pl.ds(base, RC)])
            pltpu.sync_copy(g0b, g0_hbm.at[pl.ds(base, RC)])


@functools.partial(
    pl.kernel,
    out_type=(
        jax.ShapeDtypeStruct((NPAD, D), _f32),
        jax.ShapeDtypeStruct((NPAD, D), _f32),
    ),
    mesh=_mesh,
    scratch_types=[
        pltpu.VMEM((2, PG, CW), jnp.int32),
        pltpu.VMEM((2, PG, CW), jnp.int32),
        pltpu.VMEM((CW, D), _f32),
        pltpu.VMEM((CW, D), _f32),
        pltpu.SemaphoreType.DMA,
        pltpu.SemaphoreType.DMA,
        pltpu.SemaphoreType.DMA,
        pltpu.SemaphoreType.DMA,
        pltpu.SemaphoreType.DMA,
        pltpu.SemaphoreType.DMA,
        pltpu.VMEM_SHARED((NPAD, D), _f32),
    ],
)
def _prop(g_hbm, dst_hbm, src_hbm, p0_hbm, p1_hbm,
          dstb, srcb, b0, b1, g0s, g1s, s0s, s1s, i0s, i1s, acc):
    c = lax.axis_index("c")
    s = lax.axis_index("s")
    w = c * NS + s

    _zero_rows(b0, CW)
    for k in range(RPT // RC):
        pltpu.sync_copy(b0.at[pl.ds(0, RC)],
                        acc.at[pl.ds(s * RPT + k * RC, RC)])
    plsc.subcore_barrier()

    base_row = w * NCHUNK
    bufs = (b0, b1)
    gsems = (g0s, g1s)
    ssems = (s0s, s1s)
    isems = (i0s, i1s)
    NG = NCHUNK // PG
    pend_g = [None, None]
    pend_s = [None, None]
    pend_i = [None, None]

    def load_idx(g, sync):
        slot = g % 2
        r0 = base_row + g * PG
        if sync:
            pltpu.sync_copy(dst_hbm.at[pl.ds(r0, PG)], dstb.at[slot])
            pltpu.sync_copy(src_hbm.at[pl.ds(r0, PG)], srcb.at[slot])
        else:
            pend_i[slot] = (
                pltpu.async_copy(dst_hbm.at[pl.ds(r0, PG)], dstb.at[slot],
                                 isems[slot]),
                pltpu.async_copy(src_hbm.at[pl.ds(r0, PG)], srcb.at[slot],
                                 isems[slot]),
            )

    def issue_gather(k):
        g, jj = divmod(k, PG)
        p = k % 2
        if pend_s[p] is not None:
            pend_s[p].wait()
            pend_s[p] = None
        pend_g[p] = pltpu.async_copy(
            g_hbm.at[srcb.at[g % 2, jj]], bufs[p], gsems[p])

    load_idx(0, sync=True)
    if NG > 1:
        load_idx(1, sync=False)
    issue_gather(0)
    for k in range(NCHUNK):
        g, jj = divmod(k, PG)
        p = k % 2
        if k + 1 < NCHUNK:
            g2, jj2 = divmod(k + 1, PG)
            if jj2 == 0 and pend_i[g2 % 2] is not None:
                for d in pend_i[g2 % 2]:
                    d.wait()
                pend_i[g2 % 2] = None
            issue_gather(k + 1)
        if jj == 2 and 2 <= g + 1 < NG:
            load_idx(g + 1, sync=False)
        pend_g[p].wait()
        pend_g[p] = None
        pend_s[p] = pltpu.async_copy(
            bufs[p], acc.at[dstb.at[g % 2, jj]], ssems[p], add=True)
    for p in range(2):
        if pend_s[p] is not None:
            pend_s[p].wait()
            pend_s[p] = None
    plsc.subcore_barrier()

    @pl.when(c == 0)
    def _():
        for k in range(RPT // RC):
            r = s * RPT + k * RC
            pltpu.sync_copy(acc.at[pl.ds(r, RC)], p0_hbm.at[pl.ds(r, RC)])

    @pl.when(c == 1)
    def _():
        for k in range(RPT // RC):
            r = s * RPT + k * RC
            pltpu.sync_copy(acc.at[pl.ds(r, RC)], p1_hbm.at[pl.ds(r, RC)])


@functools.partial(
    pl.kernel,
    out_type=jax.ShapeDtypeStruct((N, D), _f32),
    mesh=_mesh,
    scratch_types=[
        pltpu.VMEM((RC, D), _f32),
        pltpu.VMEM((RC, D), _f32),
        pltpu.VMEM((RC,), _f32),
    ],
)
def _scale(p0_hbm, p1_hbm, dinv_hbm, g_hbm, b0, b1, db):
    c = lax.axis_index("c")
    s = lax.axis_index("s")
    w = c * NS + s

    for t in range(4):
        kk = w + 32 * t

        @pl.when(kk < 125)
        def _():
            base = kk * RC
            pltpu.sync_copy(p0_hbm.at[pl.ds(base, RC)], b0)
            pltpu.sync_copy(p1_hbm.at[pl.ds(base, RC)], b1)
            pltpu.sync_copy(dinv_hbm.at[pl.ds(base, RC)], db)

            def row16(r16, _):
                dv = db[pl.ds(r16 * 16, 16)]
                for i in range(16):
                    d = dv[i]
                    d2 = d * d
                    r = r16 * 16 + i
                    for j in range(D // 16):
                        sl = pl.ds(j * 16, 16)
                        b0[r, sl] = d2 * (b0[r, sl] + b1[r, sl])
                return ()
            lax.fori_loop(0, RC // 16, row16, ())
            pltpu.sync_copy(b0, g_hbm.at[pl.ds(base, RC)])


def _heads_body(u, p01, p11, p02, p12, p03, p13, dinv,
                cw1, cb1, cw2, cb2, aw1, ab1, aw2, ab2, sw1, sb1, sw2, sb2,
                churn, cat, sku, uf):
    psum = (p01[...] + p11[...] + p02[...] + p12[...] + p03[...] + p13[...])
    x = (u[...] + dinv[...] * psum) * 0.25
    uf[...] = x

    def head(w1, b1, w2, b2):
        h = jnp.maximum(
            jnp.dot(x, w1[...], preferred_element_type=jnp.float32) + b1[...],
            0.0)
        return jax.nn.sigmoid(
            jnp.dot(h, w2[...], preferred_element_type=jnp.float32) + b2[...])

    churn[...] = head(cw1, cb1, cw2, cb2)
    cat[...] = head(aw1, ab1, aw2, ab2)
    sku[...] = head(sw1, sb1, sw2, sb2)


_BU = 1000


def _row_spec(cols):
    return pl.BlockSpec((_BU, cols), lambda i: (i, 0))


def _full_spec(r, cols):
    return pl.BlockSpec((r, cols), lambda i: (0, 0))


def _heads(u, p01, p11, p02, p12, p03, p13, dinv2,
           cw1, cb1, cw2, cb2, aw1, ab1, aw2, ab2, sw1, sb1, sw2, sb2):
    nu = u.shape[0]
    return pl.pallas_call(
        _heads_body,
        grid=(nu // _BU,),
        in_specs=[
            _row_spec(D),
            _row_spec(D), _row_spec(D), _row_spec(D),
            _row_spec(D), _row_spec(D), _row_spec(D),
            _row_spec(1),
            _full_spec(D, 128), _full_spec(1, 128),
            _full_spec(128, 1), _full_spec(1, 1),
            _full_spec(D, 128), _full_spec(1, 128),
            _full_spec(128, 100), _full_spec(1, 100),
            _full_spec(D, 128), _full_spec(1, 128),
            _full_spec(128, 1000), _full_spec(1, 1000),
        ],
        out_specs=[
            _row_spec(1), _row_spec(100), _row_spec(1000), _row_spec(D),
        ],
        out_shape=[
            jax.ShapeDtypeStruct((nu, 1), _f32),
            jax.ShapeDtypeStruct((nu, 100), _f32),
            jax.ShapeDtypeStruct((nu, 1000), _f32),
            jax.ShapeDtypeStruct((nu, D), _f32),
        ],
    )(u, p01, p11, p02, p12, p03, p13, dinv2,
      cw1, cb1, cw2, cb2, aw1, ab1, aw2, ab2, sw1, sb1, sw2, sb2)


def kernel(user_emb_w, item_emb_w, churn_w1, churn_b1, churn_w2, churn_b2,
           cat_w1, cat_b1, cat_w2, cat_b2, sku_w1, sku_b1, sku_w2, sku_b2,
           edge_index):
    ei = edge_index.astype(jnp.int32)
    dst = jnp.concatenate([ei[0], ei[1]]).reshape(NW * NCHUNK, CW)
    src = jnp.concatenate([ei[1], ei[0]]).reshape(NW * NCHUNK, CW)
    emb = jnp.concatenate([user_emb_w, item_emb_w], axis=0)

    dinv, g0 = _prep(dst, emb)
    p01, p11 = _prop(g0, dst, src)
    g1 = _scale(p01, p11, dinv)
    p02, p12 = _prop(g1, dst, src)
    g2 = _scale(p02, p12, dinv)
    p03, p13 = _prop(g2, dst, src)

    nu = user_emb_w.shape[0]
    dinv2 = dinv[:nu].reshape(nu, 1)
    churn, cat, sku, uf = _heads(
        user_emb_w, p01, p11, p02, p12, p03, p13, dinv2,
        churn_w1, churn_b1.reshape(1, 128), churn_w2, churn_b2.reshape(1, 1),
        cat_w1, cat_b1.reshape(1, 128), cat_w2, cat_b2.reshape(1, 100),
        sku_w1, sku_b1.reshape(1, 128), sku_w2, sku_b2.reshape(1, 1000))
    return churn, cat, sku, uf


# speedup vs baseline: 36.3301x; 1.0789x over previous
import functools

import jax
import jax.numpy as jnp
from jax import lax
from jax.experimental import pallas as pl
from jax.experimental.pallas import tpu as pltpu
from jax.experimental.pallas import tpu_sc as plsc

N = 10000
D = 128
E2 = 640000
NC = 2
NS = 16
NW = NC * NS
M = E2 // NW
CW = 125
NCHUNK = M // CW
IG = 8
PG = 16
NPAD = 10240
RPT = NPAD // NS
RC = 80

_mesh = plsc.VectorSubcoreMesh(core_axis_name="c", subcore_axis_name="s")
_f32 = jnp.float32


def _zero_rows(buf, nrows):
    def row(r, _):
        for j in range(D // 16):
            buf[r, pl.ds(j * 16, 16)] = jnp.zeros((16,), _f32)
        return ()
    lax.fori_loop(0, nrows, row, ())


@functools.partial(
    pl.kernel,
    out_type=jax.ShapeDtypeStruct((NPAD,), _f32),
    mesh=_mesh,
    scratch_types=[
        pltpu.VMEM((IG, CW), jnp.int32),
        pltpu.VMEM((128,), _f32),
        pltpu.VMEM((640,), _f32),
        pltpu.VMEM_SHARED((NPAD,), _f32),
    ],
)
def _hist_kernel(dst_hbm, hist_hbm, idxb, ones, zb, hist):
    c = lax.axis_index("c")
    s = lax.axis_index("s")

    for i in range(40):
        zb[pl.ds(i * 16, 16)] = jnp.zeros((16,), _f32)
    for i in range(8):
        ones[pl.ds(i * 16, 16)] = jnp.full((16,), 1.0, _f32)
    pltpu.sync_copy(zb, hist.at[pl.ds(s * 640, 640)])
    plsc.subcore_barrier()

    base_row = s * 2 * NCHUNK

    def hbody(j8, _):
        pltpu.sync_copy(dst_hbm.at[pl.ds(base_row + j8 * IG, IG)], idxb)
        for jj in range(IG):
            pltpu.sync_copy(ones.at[pl.ds(0, CW)], hist.at[idxb.at[jj]],
                            add=True)
        return ()
    lax.fori_loop(0, 2 * NCHUNK // IG, hbody, ())
    plsc.subcore_barrier()

    @pl.when(c == 0)
    def _():
        pltpu.sync_copy(hist.at[pl.ds(s * 640, 640)],
                        hist_hbm.at[pl.ds(s * 640, 640)])


def _dg_body(hist, emb, dinv, g0):
    h = hist[...]
    d = jnp.where(h > 0.5, lax.rsqrt(jnp.maximum(h, 1.0)), 0.0)
    dinv[...] = d
    g0[...] = d * emb[...]


def _dinv_g0(hist2, emb):
    return pl.pallas_call(
        _dg_body,
        grid=(N // _BU,),
        in_specs=[_row_spec(1), _row_spec(D)],
        out_specs=[_row_spec(1), _row_spec(D)],
        out_shape=[
            jax.ShapeDtypeStruct((N, 1), _f32),
            jax.ShapeDtypeStruct((N, D), _f32),
        ],
    )(hist2, emb)


@functools.partial(
    pl.kernel,
    out_type=(
        jax.ShapeDtypeStruct((NPAD, D), _f32),
        jax.ShapeDtypeStruct((NPAD, D), _f32),
    ),
    mesh=_mesh,
    scratch_types=[
        pltpu.VMEM((2, PG, CW), jnp.int32),
        pltpu.VMEM((2, PG, CW), jnp.int32),
        pltpu.VMEM((CW, D), _f32),
        pltpu.VMEM((CW, D), _f32),
        pltpu.SemaphoreType.DMA,
        pltpu.SemaphoreType.DMA,
        pltpu.SemaphoreType.DMA,
        pltpu.SemaphoreType.DMA,
        pltpu.SemaphoreType.DMA,
        pltpu.SemaphoreType.DMA,
        pltpu.VMEM_SHARED((NPAD, D), _f32),
    ],
)
def _prop(g_hbm, dst_hbm, src_hbm, p0_hbm, p1_hbm,
          dstb, srcb, b0, b1, g0s, g1s, s0s, s1s, i0s, i1s, acc):
    c = lax.axis_index("c")
    s = lax.axis_index("s")
    w = c * NS + s

    _zero_rows(b0, CW)
    for k in range(RPT // RC):
        pltpu.sync_copy(b0.at[pl.ds(0, RC)],
                        acc.at[pl.ds(s * RPT + k * RC, RC)])
    plsc.subcore_barrier()

    base_row = w * NCHUNK
    bufs = (b0, b1)
    gsems = (g0s, g1s)
    ssems = (s0s, s1s)
    isems = (i0s, i1s)
    NG = NCHUNK // PG
    pend_g = [None, None]
    pend_s = [None, None]
    pend_i = [None, None]

    def load_idx(g, sync):
        slot = g % 2
        r0 = base_row + g * PG
        if sync:
            pltpu.sync_copy(dst_hbm.at[pl.ds(r0, PG)], dstb.at[slot])
            pltpu.sync_copy(src_hbm.at[pl.ds(r0, PG)], srcb.at[slot])
        else:
            pend_i[slot] = (
                pltpu.async_copy(dst_hbm.at[pl.ds(r0, PG)], dstb.at[slot],
                                 isems[slot]),
                pltpu.async_copy(src_hbm.at[pl.ds(r0, PG)], srcb.at[slot],
                                 isems[slot]),
            )

    def issue_gather(k):
        g, jj = divmod(k, PG)
        p = k % 2
        if pend_s[p] is not None:
            pend_s[p].wait()
            pend_s[p] = None
        pend_g[p] = pltpu.async_copy(
            g_hbm.at[srcb.at[g % 2, jj]], bufs[p], gsems[p])

    load_idx(0, sync=True)
    if NG > 1:
        load_idx(1, sync=False)
    issue_gather(0)
    for k in range(NCHUNK):
        g, jj = divmod(k, PG)
        p = k % 2
        if k + 1 < NCHUNK:
            g2, jj2 = divmod(k + 1, PG)
            if jj2 == 0 and pend_i[g2 % 2] is not None:
                for d in pend_i[g2 % 2]:
                    d.wait()
                pend_i[g2 % 2] = None
            issue_gather(k + 1)
        if jj == 2 and 2 <= g + 1 < NG:
            load_idx(g + 1, sync=False)
        pend_g[p].wait()
        pend_g[p] = None
        pend_s[p] = pltpu.async_copy(
            bufs[p], acc.at[dstb.at[g % 2, jj]], ssems[p], add=True)
    for p in range(2):
        if pend_s[p] is not None:
            pend_s[p].wait()
            pend_s[p] = None
    plsc.subcore_barrier()

    @pl.when(c == 0)
    def _():
        for k in range(RPT // RC):
            r = s * RPT + k * RC
            pltpu.sync_copy(acc.at[pl.ds(r, RC)], p0_hbm.at[pl.ds(r, RC)])

    @pl.when(c == 1)
    def _():
        for k in range(RPT // RC):
            r = s * RPT + k * RC
            pltpu.sync_copy(acc.at[pl.ds(r, RC)], p1_hbm.at[pl.ds(r, RC)])


def _scale_body(p0, p1, dinv, g):
    d = dinv[...]
    g[...] = (d * d) * (p0[...] + p1[...])


def _scale(p0, p1, dinv2):
    return pl.pallas_call(
        _scale_body,
        grid=(N // _BU,),
        in_specs=[_row_spec(D), _row_spec(D), _row_spec(1)],
        out_specs=_row_spec(D),
        out_shape=jax.ShapeDtypeStruct((N, D), _f32),
    )(p0, p1, dinv2)


def _heads_body(u, p01, p11, p02, p12, p03, p13, dinv,
                cw1, cb1, cw2, cb2, aw1, ab1, aw2, ab2, sw1, sb1, sw2, sb2,
                churn, cat, sku, uf):
    psum = (p01[...] + p11[...] + p02[...] + p12[...] + p03[...] + p13[...])
    x = (u[...] + dinv[...] * psum) * 0.25
    uf[...] = x

    def head(w1, b1, w2, b2):
        h = jnp.maximum(
            jnp.dot(x, w1[...], preferred_element_type=jnp.float32) + b1[...],
            0.0)
        return jax.nn.sigmoid(
            jnp.dot(h, w2[...], preferred_element_type=jnp.float32) + b2[...])

    churn[...] = head(cw1, cb1, cw2, cb2)
    cat[...] = head(aw1, ab1, aw2, ab2)
    sku[...] = head(sw1, sb1, sw2, sb2)


_BU = 1000


def _row_spec(cols):
    return pl.BlockSpec((_BU, cols), lambda i: (i, 0))


def _full_spec(r, cols):
    return pl.BlockSpec((r, cols), lambda i: (0, 0))


def _heads(u, p01, p11, p02, p12, p03, p13, dinv2,
           cw1, cb1, cw2, cb2, aw1, ab1, aw2, ab2, sw1, sb1, sw2, sb2):
    nu = u.shape[0]
    return pl.pallas_call(
        _heads_body,
        grid=(nu // _BU,),
        in_specs=[
            _row_spec(D),
            _row_spec(D), _row_spec(D), _row_spec(D),
            _row_spec(D), _row_spec(D), _row_spec(D),
            _row_spec(1),
            _full_spec(D, 128), _full_spec(1, 128),
            _full_spec(128, 1), _full_spec(1, 1),
            _full_spec(D, 128), _full_spec(1, 128),
            _full_spec(128, 100), _full_spec(1, 100),
            _full_spec(D, 128), _full_spec(1, 128),
            _full_spec(128, 1000), _full_spec(1, 1000),
        ],
        out_specs=[
            _row_spec(1), _row_spec(100), _row_spec(1000), _row_spec(D),
        ],
        out_shape=[
            jax.ShapeDtypeStruct((nu, 1), _f32),
            jax.ShapeDtypeStruct((nu, 100), _f32),
            jax.ShapeDtypeStruct((nu, 1000), _f32),
            jax.ShapeDtypeStruct((nu, D), _f32),
        ],
    )(u, p01, p11, p02, p12, p03, p13, dinv2,
      cw1, cb1, cw2, cb2, aw1, ab1, aw2, ab2, sw1, sb1, sw2, sb2)


def kernel(user_emb_w, item_emb_w, churn_w1, churn_b1, churn_w2, churn_b2,
           cat_w1, cat_b1, cat_w2, cat_b2, sku_w1, sku_b1, sku_w2, sku_b2,
           edge_index):
    ei = edge_index.astype(jnp.int32)
    dst = jnp.concatenate([ei[0], ei[1]]).reshape(NW * NCHUNK, CW)
    src = jnp.concatenate([ei[1], ei[0]]).reshape(NW * NCHUNK, CW)
    emb = jnp.concatenate([user_emb_w, item_emb_w], axis=0)

    hist = _hist_kernel(dst)
    dinv2f, g0 = _dinv_g0(hist[:N].reshape(N, 1), emb)
    p01, p11 = _prop(g0, dst, src)
    g1 = _scale(p01, p11, dinv2f)
    p02, p12 = _prop(g1, dst, src)
    g2 = _scale(p02, p12, dinv2f)
    p03, p13 = _prop(g2, dst, src)

    nu = user_emb_w.shape[0]
    dinv2 = dinv2f[:nu]
    churn, cat, sku, uf = _heads(
        user_emb_w, p01, p11, p02, p12, p03, p13, dinv2,
        churn_w1, churn_b1.reshape(1, 128), churn_w2, churn_b2.reshape(1, 1),
        cat_w1, cat_b1.reshape(1, 128), cat_w2, cat_b2.reshape(1, 100),
        sku_w1, sku_b1.reshape(1, 128), sku_w2, sku_b2.reshape(1, 1000))
    return churn, cat, sku, uf


# speedup vs baseline: 37.4098x; 1.0297x over previous
import functools

import jax
import jax.numpy as jnp
from jax import lax
from jax.experimental import pallas as pl
from jax.experimental.pallas import tpu as pltpu
from jax.experimental.pallas import tpu_sc as plsc

N = 10000
D = 128
E2 = 640000
NC = 2
NS = 16
NW = NC * NS
M = E2 // NW
CW = 125
NCHUNK = M // CW
IG = 8
PG = 16
NPAD = 10240
RPT = NPAD // NS
RC = 80

_mesh = plsc.VectorSubcoreMesh(core_axis_name="c", subcore_axis_name="s")
_f32 = jnp.float32


def _zero_rows(buf, nrows):
    def row(r, _):
        for j in range(D // 16):
            buf[r, pl.ds(j * 16, 16)] = jnp.zeros((16,), _f32)
        return ()
    lax.fori_loop(0, nrows, row, ())


@functools.partial(
    pl.kernel,
    out_type=jax.ShapeDtypeStruct((2 * NPAD,), _f32),
    mesh=_mesh,
    scratch_types=[
        pltpu.VMEM((IG, CW), jnp.int32),
        pltpu.VMEM((128,), _f32),
        pltpu.VMEM((640,), _f32),
        pltpu.VMEM_SHARED((NPAD,), _f32),
    ],
)
def _hist_kernel(dst_hbm, hist_hbm, idxb, ones, zb, hist):
    c = lax.axis_index("c")
    s = lax.axis_index("s")

    for i in range(40):
        zb[pl.ds(i * 16, 16)] = jnp.zeros((16,), _f32)
    for i in range(8):
        ones[pl.ds(i * 16, 16)] = jnp.full((16,), 1.0, _f32)
    pltpu.sync_copy(zb, hist.at[pl.ds(s * 640, 640)])
    plsc.subcore_barrier()

    base_row = (c * NS + s) * NCHUNK

    def hbody(j8, _):
        pltpu.sync_copy(dst_hbm.at[pl.ds(base_row + j8 * IG, IG)], idxb)
        for jj in range(IG):
            pltpu.sync_copy(ones.at[pl.ds(0, CW)], hist.at[idxb.at[jj]],
                            add=True)
        return ()
    lax.fori_loop(0, NCHUNK // IG, hbody, ())
    plsc.subcore_barrier()

    pltpu.sync_copy(hist.at[pl.ds(s * 640, 640)],
                    hist_hbm.at[pl.ds(c * NPAD + s * 640, 640)])


def _dg_body(h0, h1, emb, dinv, g0):
    h = h0[...] + h1[...]
    d = jnp.where(h > 0.5, lax.rsqrt(jnp.maximum(h, 1.0)), 0.0)
    dinv[...] = d
    g0[...] = d * emb[...]


def _dinv_g0(hist0, hist1, emb):
    return pl.pallas_call(
        _dg_body,
        grid=(N // _BU,),
        in_specs=[_row_spec(1), _row_spec(1), _row_spec(D)],
        out_specs=[_row_spec(1), _row_spec(D)],
        out_shape=[
            jax.ShapeDtypeStruct((N, 1), _f32),
            jax.ShapeDtypeStruct((N, D), _f32),
        ],
    )(hist0, hist1, emb)


@functools.partial(
    pl.kernel,
    out_type=(
        jax.ShapeDtypeStruct((NPAD, D), _f32),
        jax.ShapeDtypeStruct((NPAD, D), _f32),
    ),
    mesh=_mesh,
    scratch_types=[
        pltpu.VMEM((2, PG, CW), jnp.int32),
        pltpu.VMEM((2, PG, CW), jnp.int32),
        pltpu.VMEM((CW, D), _f32),
        pltpu.VMEM((CW, D), _f32),
        pltpu.SemaphoreType.DMA,
        pltpu.SemaphoreType.DMA,
        pltpu.SemaphoreType.DMA,
        pltpu.SemaphoreType.DMA,
        pltpu.SemaphoreType.DMA,
        pltpu.SemaphoreType.DMA,
        pltpu.VMEM_SHARED((NPAD, D), _f32),
    ],
)
def _prop(g_hbm, dst_hbm, src_hbm, p0_hbm, p1_hbm,
          dstb, srcb, b0, b1, g0s, g1s, s0s, s1s, i0s, i1s, acc):
    c = lax.axis_index("c")
    s = lax.axis_index("s")
    w = c * NS + s

    _zero_rows(b0, CW)
    for k in range(RPT // RC):
        pltpu.sync_copy(b0.at[pl.ds(0, RC)],
                        acc.at[pl.ds(s * RPT + k * RC, RC)])
    plsc.subcore_barrier()

    base_row = w * NCHUNK
    bufs = (b0, b1)
    gsems = (g0s, g1s)
    ssems = (s0s, s1s)
    isems = (i0s, i1s)
    NG = NCHUNK // PG
    pend_g = [None, None]
    pend_s = [None, None]
    pend_i = [None, None]

    def load_idx(g, sync):
        slot = g % 2
        r0 = base_row + g * PG
        if sync:
            pltpu.sync_copy(dst_hbm.at[pl.ds(r0, PG)], dstb.at[slot])
            pltpu.sync_copy(src_hbm.at[pl.ds(r0, PG)], srcb.at[slot])
        else:
            pend_i[slot] = (
                pltpu.async_copy(dst_hbm.at[pl.ds(r0, PG)], dstb.at[slot],
                                 isems[slot]),
                pltpu.async_copy(src_hbm.at[pl.ds(r0, PG)], srcb.at[slot],
                                 isems[slot]),
            )

    def issue_gather(k):
        g, jj = divmod(k, PG)
        p = k % 2
        if pend_s[p] is not None:
            pend_s[p].wait()
            pend_s[p] = None
        pend_g[p] = pltpu.async_copy(
            g_hbm.at[srcb.at[g % 2, jj]], bufs[p], gsems[p])

    load_idx(0, sync=True)
    if NG > 1:
        load_idx(1, sync=False)
    issue_gather(0)
    for k in range(NCHUNK):
        g, jj = divmod(k, PG)
        p = k % 2
        if k + 1 < NCHUNK:
            g2, jj2 = divmod(k + 1, PG)
            if jj2 == 0 and pend_i[g2 % 2] is not None:
                for d in pend_i[g2 % 2]:
                    d.wait()
                pend_i[g2 % 2] = None
            issue_gather(k + 1)
        if jj == 2 and 2 <= g + 1 < NG:
            load_idx(g + 1, sync=False)
        pend_g[p].wait()
        pend_g[p] = None
        pend_s[p] = pltpu.async_copy(
            bufs[p], acc.at[dstb.at[g % 2, jj]], ssems[p], add=True)
    for p in range(2):
        if pend_s[p] is not None:
            pend_s[p].wait()
            pend_s[p] = None
    plsc.subcore_barrier()

    @pl.when(c == 0)
    def _():
        for k in range(RPT // RC):
            r = s * RPT + k * RC
            pltpu.sync_copy(acc.at[pl.ds(r, RC)], p0_hbm.at[pl.ds(r, RC)])

    @pl.when(c == 1)
    def _():
        for k in range(RPT // RC):
            r = s * RPT + k * RC
            pltpu.sync_copy(acc.at[pl.ds(r, RC)], p1_hbm.at[pl.ds(r, RC)])


def _scale_body(p0, p1, dinv, g):
    d = dinv[...]
    g[...] = (d * d) * (p0[...] + p1[...])


def _scale(p0, p1, dinv2):
    return pl.pallas_call(
        _scale_body,
        grid=(N // _BU,),
        in_specs=[_row_spec(D), _row_spec(D), _row_spec(1)],
        out_specs=_row_spec(D),
        out_shape=jax.ShapeDtypeStruct((N, D), _f32),
    )(p0, p1, dinv2)


def _heads_body(u, p01, p11, p02, p12, p03, p13, dinv,
                cw1, cb1, cw2, cb2, aw1, ab1, aw2, ab2, sw1, sb1, sw2, sb2,
                churn, cat, sku, uf):
    psum = (p01[...] + p11[...] + p02[...] + p12[...] + p03[...] + p13[...])
    x = (u[...] + dinv[...] * psum) * 0.25
    uf[...] = x

    def head(w1, b1, w2, b2):
        h = jnp.maximum(
            jnp.dot(x, w1[...], preferred_element_type=jnp.float32) + b1[...],
            0.0)
        return jax.nn.sigmoid(
            jnp.dot(h, w2[...], preferred_element_type=jnp.float32) + b2[...])

    churn[...] = head(cw1, cb1, cw2, cb2)
    cat[...] = head(aw1, ab1, aw2, ab2)
    sku[...] = head(sw1, sb1, sw2, sb2)


_BU = 1000


def _row_spec(cols):
    return pl.BlockSpec((_BU, cols), lambda i: (i, 0))


def _full_spec(r, cols):
    return pl.BlockSpec((r, cols), lambda i: (0, 0))


def _heads(u, p01, p11, p02, p12, p03, p13, dinv2,
           cw1, cb1, cw2, cb2, aw1, ab1, aw2, ab2, sw1, sb1, sw2, sb2):
    nu = u.shape[0]
    return pl.pallas_call(
        _heads_body,
        grid=(nu // _BU,),
        in_specs=[
            _row_spec(D),
            _row_spec(D), _row_spec(D), _row_spec(D),
            _row_spec(D), _row_spec(D), _row_spec(D),
            _row_spec(1),
            _full_spec(D, 128), _full_spec(1, 128),
            _full_spec(128, 1), _full_spec(1, 1),
            _full_spec(D, 128), _full_spec(1, 128),
            _full_spec(128, 100), _full_spec(1, 100),
            _full_spec(D, 128), _full_spec(1, 128),
            _full_spec(128, 1000), _full_spec(1, 1000),
        ],
        out_specs=[
            _row_spec(1), _row_spec(100), _row_spec(1000), _row_spec(D),
        ],
        out_shape=[
            jax.ShapeDtypeStruct((nu, 1), _f32),
            jax.ShapeDtypeStruct((nu, 100), _f32),
            jax.ShapeDtypeStruct((nu, 1000), _f32),
            jax.ShapeDtypeStruct((nu, D), _f32),
        ],
    )(u, p01, p11, p02, p12, p03, p13, dinv2,
      cw1, cb1, cw2, cb2, aw1, ab1, aw2, ab2, sw1, sb1, sw2, sb2)


def kernel(user_emb_w, item_emb_w, churn_w1, churn_b1, churn_w2, churn_b2,
           cat_w1, cat_b1, cat_w2, cat_b2, sku_w1, sku_b1, sku_w2, sku_b2,
           edge_index):
    ei = edge_index.astype(jnp.int32)
    dst = jnp.concatenate([ei[0], ei[1]]).reshape(NW * NCHUNK, CW)
    src = jnp.concatenate([ei[1], ei[0]]).reshape(NW * NCHUNK, CW)
    emb = jnp.concatenate([user_emb_w, item_emb_w], axis=0)

    hist = _hist_kernel(dst)
    dinv2f, g0 = _dinv_g0(hist[:N].reshape(N, 1),
                          hist[NPAD:NPAD + N].reshape(N, 1), emb)
    p01, p11 = _prop(g0, dst, src)
    g1 = _scale(p01, p11, dinv2f)
    p02, p12 = _prop(g1, dst, src)
    g2 = _scale(p02, p12, dinv2f)
    p03, p13 = _prop(g2, dst, src)

    nu = user_emb_w.shape[0]
    dinv2 = dinv2f[:nu]
    churn, cat, sku, uf = _heads(
        user_emb_w, p01, p11, p02, p12, p03, p13, dinv2,
        churn_w1, churn_b1.reshape(1, 128), churn_w2, churn_b2.reshape(1, 1),
        cat_w1, cat_b1.reshape(1, 128), cat_w2, cat_b2.reshape(1, 100),
        sku_w1, sku_b1.reshape(1, 128), sku_w2, sku_b2.reshape(1, 1000))
    return churn, cat, sku, uf


# speedup vs baseline: 40.9741x; 1.0953x over previous
import functools

import jax
import jax.numpy as jnp
from jax import lax
from jax.experimental import pallas as pl
from jax.experimental.pallas import tpu as pltpu
from jax.experimental.pallas import tpu_sc as plsc

N = 10000
D = 128
E2 = 640000
NC = 2
NS = 16
NW = NC * NS
M = E2 // NW
CW = 100
NCHUNK = M // CW
NB = 3
IG = 8
PG = 8
NPAD = 10240
RPT = NPAD // NS
RC = 80

_mesh = plsc.VectorSubcoreMesh(core_axis_name="c", subcore_axis_name="s")
_f32 = jnp.float32


def _zero_rows(buf, nrows):
    def row(r, _):
        for j in range(D // 16):
            buf[r, pl.ds(j * 16, 16)] = jnp.zeros((16,), _f32)
        return ()
    lax.fori_loop(0, nrows, row, ())


@functools.partial(
    pl.kernel,
    out_type=jax.ShapeDtypeStruct((2 * NPAD,), _f32),
    mesh=_mesh,
    scratch_types=[
        pltpu.VMEM((IG, CW), jnp.int32),
        pltpu.VMEM((128,), _f32),
        pltpu.VMEM((640,), _f32),
        pltpu.VMEM_SHARED((NPAD,), _f32),
    ],
)
def _hist_kernel(dst_hbm, hist_hbm, idxb, ones, zb, hist):
    c = lax.axis_index("c")
    s = lax.axis_index("s")

    for i in range(40):
        zb[pl.ds(i * 16, 16)] = jnp.zeros((16,), _f32)
    for i in range(8):
        ones[pl.ds(i * 16, 16)] = jnp.full((16,), 1.0, _f32)
    pltpu.sync_copy(zb, hist.at[pl.ds(s * 640, 640)])
    plsc.subcore_barrier()

    base_row = (c * NS + s) * NCHUNK

    def hbody(j8, _):
        pltpu.sync_copy(dst_hbm.at[pl.ds(base_row + j8 * IG, IG)], idxb)
        for jj in range(IG):
            pltpu.sync_copy(ones.at[pl.ds(0, CW)], hist.at[idxb.at[jj]],
                            add=True)
        return ()
    lax.fori_loop(0, NCHUNK // IG, hbody, ())
    plsc.subcore_barrier()

    pltpu.sync_copy(hist.at[pl.ds(s * 640, 640)],
                    hist_hbm.at[pl.ds(c * NPAD + s * 640, 640)])


def _dg_body(h0, h1, emb, dinv, g0):
    h = h0[...] + h1[...]
    d = jnp.where(h > 0.5, lax.rsqrt(jnp.maximum(h, 1.0)), 0.0)
    dinv[...] = d
    g0[...] = d * emb[...]


def _dinv_g0(hist0, hist1, emb):
    return pl.pallas_call(
        _dg_body,
        grid=(N // _BU,),
        in_specs=[_row_spec(1), _row_spec(1), _row_spec(D)],
        out_specs=[_row_spec(1), _row_spec(D)],
        out_shape=[
            jax.ShapeDtypeStruct((N, 1), _f32),
            jax.ShapeDtypeStruct((N, D), _f32),
        ],
    )(hist0, hist1, emb)


@functools.partial(
    pl.kernel,
    out_type=(
        jax.ShapeDtypeStruct((NPAD, D), _f32),
        jax.ShapeDtypeStruct((NPAD, D), _f32),
    ),
    mesh=_mesh,
    scratch_types=[
        pltpu.VMEM((2, PG, CW), jnp.int32),
        pltpu.VMEM((2, PG, CW), jnp.int32),
        pltpu.VMEM((CW, D), _f32),
        pltpu.VMEM((CW, D), _f32),
        pltpu.VMEM((CW, D), _f32),
        pltpu.SemaphoreType.DMA,
        pltpu.SemaphoreType.DMA,
        pltpu.SemaphoreType.DMA,
        pltpu.SemaphoreType.DMA,
        pltpu.SemaphoreType.DMA,
        pltpu.SemaphoreType.DMA,
        pltpu.SemaphoreType.DMA,
        pltpu.SemaphoreType.DMA,
        pltpu.VMEM_SHARED((NPAD, D), _f32),
    ],
)
def _prop(g_hbm, dst_hbm, src_hbm, p0_hbm, p1_hbm,
          dstb, srcb, b0, b1, b2, g0s, g1s, g2s, s0s, s1s, s2s, i0s, i1s,
          acc):
    c = lax.axis_index("c")
    s = lax.axis_index("s")
    w = c * NS + s

    _zero_rows(b0, CW)
    for k in range(RPT // RC):
        pltpu.sync_copy(b0.at[pl.ds(0, RC)],
                        acc.at[pl.ds(s * RPT + k * RC, RC)])
    plsc.subcore_barrier()

    base_row = w * NCHUNK
    bufs = (b0, b1, b2)
    gsems = (g0s, g1s, g2s)
    ssems = (s0s, s1s, s2s)
    isems = (i0s, i1s)
    NG = NCHUNK // PG
    pend_g = [None] * NB
    pend_s = [None] * NB
    pend_i = [None, None]

    def load_idx(g, sync):
        slot = g % 2
        r0 = base_row + g * PG
        if sync:
            pltpu.sync_copy(dst_hbm.at[pl.ds(r0, PG)], dstb.at[slot])
            pltpu.sync_copy(src_hbm.at[pl.ds(r0, PG)], srcb.at[slot])
        else:
            pend_i[slot] = (
                pltpu.async_copy(dst_hbm.at[pl.ds(r0, PG)], dstb.at[slot],
                                 isems[slot]),
                pltpu.async_copy(src_hbm.at[pl.ds(r0, PG)], srcb.at[slot],
                                 isems[slot]),
            )

    def issue_gather(j):
        g, jj = divmod(j, PG)
        p = j % NB
        if pend_s[p] is not None:
            pend_s[p].wait()
            pend_s[p] = None
        if jj == 0 and pend_i[g % 2] is not None:
            for d in pend_i[g % 2]:
                d.wait()
            pend_i[g % 2] = None
        pend_g[p] = pltpu.async_copy(
            g_hbm.at[srcb.at[g % 2, jj]], bufs[p], gsems[p])

    load_idx(0, sync=True)
    if NG > 1:
        load_idx(1, sync=False)
    for j in range(NB - 1):
        issue_gather(j)
    for k in range(NCHUNK):
        g, jj = divmod(k, PG)
        p = k % NB
        if k + NB - 1 < NCHUNK:
            issue_gather(k + NB - 1)
        if jj == 2 and 2 <= g + 1 < NG:
            load_idx(g + 1, sync=False)
        pend_g[p].wait()
        pend_g[p] = None
        pend_s[p] = pltpu.async_copy(
            bufs[p], acc.at[dstb.at[g % 2, jj]], ssems[p], add=True)
    for p in range(NB):
        if pend_s[p] is not None:
            pend_s[p].wait()
            pend_s[p] = None
    plsc.subcore_barrier()

    @pl.when(c == 0)
    def _():
        for k in range(RPT // RC):
            r = s * RPT + k * RC
            pltpu.sync_copy(acc.at[pl.ds(r, RC)], p0_hbm.at[pl.ds(r, RC)])

    @pl.when(c == 1)
    def _():
        for k in range(RPT // RC):
            r = s * RPT + k * RC
            pltpu.sync_copy(acc.at[pl.ds(r, RC)], p1_hbm.at[pl.ds(r, RC)])


def _scale_body(p0, p1, dinv, g):
    d = dinv[...]
    g[...] = (d * d) * (p0[...] + p1[...])


def _scale(p0, p1, dinv2):
    return pl.pallas_call(
        _scale_body,
        grid=(N // _BU,),
        in_specs=[_row_spec(D), _row_spec(D), _row_spec(1)],
        out_specs=_row_spec(D),
        out_shape=jax.ShapeDtypeStruct((N, D), _f32),
    )(p0, p1, dinv2)


def _heads_body(u, p01, p11, p02, p12, p03, p13, dinv,
                cw1, cb1, cw2, cb2, aw1, ab1, aw2, ab2, sw1, sb1, sw2, sb2,
                churn, cat, sku, uf):
    psum = (p01[...] + p11[...] + p02[...] + p12[...] + p03[...] + p13[...])
    x = (u[...] + dinv[...] * psum) * 0.25
    uf[...] = x

    def head(w1, b1, w2, b2):
        h = jnp.maximum(
            jnp.dot(x, w1[...], preferred_element_type=jnp.float32) + b1[...],
            0.0)
        return jax.nn.sigmoid(
            jnp.dot(h, w2[...], preferred_element_type=jnp.float32) + b2[...])

    churn[...] = head(cw1, cb1, cw2, cb2)
    cat[...] = head(aw1, ab1, aw2, ab2)
    sku[...] = head(sw1, sb1, sw2, sb2)


_BU = 1000


def _row_spec(cols):
    return pl.BlockSpec((_BU, cols), lambda i: (i, 0))


def _full_spec(r, cols):
    return pl.BlockSpec((r, cols), lambda i: (0, 0))


def _heads(u, p01, p11, p02, p12, p03, p13, dinv2,
           cw1, cb1, cw2, cb2, aw1, ab1, aw2, ab2, sw1, sb1, sw2, sb2):
    nu = u.shape[0]
    return pl.pallas_call(
        _heads_body,
        grid=(nu // _BU,),
        in_specs=[
            _row_spec(D),
            _row_spec(D), _row_spec(D), _row_spec(D),
            _row_spec(D), _row_spec(D), _row_spec(D),
            _row_spec(1),
            _full_spec(D, 128), _full_spec(1, 128),
            _full_spec(128, 1), _full_spec(1, 1),
            _full_spec(D, 128), _full_spec(1, 128),
            _full_spec(128, 100), _full_spec(1, 100),
            _full_spec(D, 128), _full_spec(1, 128),
            _full_spec(128, 1000), _full_spec(1, 1000),
        ],
        out_specs=[
            _row_spec(1), _row_spec(100), _row_spec(1000), _row_spec(D),
        ],
        out_shape=[
            jax.ShapeDtypeStruct((nu, 1), _f32),
            jax.ShapeDtypeStruct((nu, 100), _f32),
            jax.ShapeDtypeStruct((nu, 1000), _f32),
            jax.ShapeDtypeStruct((nu, D), _f32),
        ],
    )(u, p01, p11, p02, p12, p03, p13, dinv2,
      cw1, cb1, cw2, cb2, aw1, ab1, aw2, ab2, sw1, sb1, sw2, sb2)


def kernel(user_emb_w, item_emb_w, churn_w1, churn_b1, churn_w2, churn_b2,
           cat_w1, cat_b1, cat_w2, cat_b2, sku_w1, sku_b1, sku_w2, sku_b2,
           edge_index):
    ei = edge_index.astype(jnp.int32)
    dst = jnp.concatenate([ei[0], ei[1]]).reshape(NW * NCHUNK, CW)
    src = jnp.concatenate([ei[1], ei[0]]).reshape(NW * NCHUNK, CW)
    emb = jnp.concatenate([user_emb_w, item_emb_w], axis=0)

    hist = _hist_kernel(dst)
    dinv2f, g0 = _dinv_g0(hist[:N].reshape(N, 1),
                          hist[NPAD:NPAD + N].reshape(N, 1), emb)
    p01, p11 = _prop(g0, dst, src)
    g1 = _scale(p01, p11, dinv2f)
    p02, p12 = _prop(g1, dst, src)
    g2 = _scale(p02, p12, dinv2f)
    p03, p13 = _prop(g2, dst, src)

    nu = user_emb_w.shape[0]
    dinv2 = dinv2f[:nu]
    churn, cat, sku, uf = _heads(
        user_emb_w, p01, p11, p02, p12, p03, p13, dinv2,
        churn_w1, churn_b1.reshape(1, 128), churn_w2, churn_b2.reshape(1, 1),
        cat_w1, cat_b1.reshape(1, 128), cat_w2, cat_b2.reshape(1, 100),
        sku_w1, sku_b1.reshape(1, 128), sku_w2, sku_b2.reshape(1, 1000))
    return churn, cat, sku, uf
